# baseline (speedup 1.0000x reference)
import functools

import jax
import jax.numpy as jnp
from jax import lax
from jax.experimental import pallas as pl
from jax.experimental.pallas import tpu as pltpu
from jax.experimental.pallas import tpu_sc as plsc

_B, _T, _C, _K = 16, 2048, 256, 1024
_NW = 32
_RW = _B * _T // _NW
_CH = 128
_NCH = _RW // _CH


def _tc_a(enc_ref, dec_ref, wpre_ref, bpre_ref, wpost_ref, bpost_ref,
          cb_ref,
          y_ref, idx_ref, cm_ref, kl_ref, cwout_ref,
          c2b_ref):
    b = pl.program_id(0)

    @pl.when(b == 0)
    def _init_consts():
        cb = cb_ref[...]
        c2col = jnp.sum(cb * cb, axis=1, keepdims=True)
        c2b_ref[...] = jnp.broadcast_to(c2col * 0.5, (_K, _T))
        cwout_ref[...] = jnp.dot(cb, wpost_ref[...],
                                 preferred_element_type=jnp.float32)

    x = enc_ref[0] - dec_ref[0]
    r = jnp.dot(x, wpre_ref[...],
                preferred_element_type=jnp.float32) + bpre_ref[...]
    dots_t = jax.lax.dot_general(cb_ref[...], r, (((1,), (1,)), ((), ())),
                                 preferred_element_type=jnp.float32)
    m_t = dots_t - c2b_ref[...]

    idxr = jnp.argmax(m_t, axis=0)
    maxm = jnp.max(m_t, axis=0, keepdims=True)
    rsq = r * r
    r2 = jnp.dot(rsq, jnp.ones((_C, 1), jnp.float32),
                 preferred_element_type=jnp.float32)
    idx_ref[0, 0, :] = idxr

    krow = jax.lax.broadcasted_iota(jnp.int32, (_K, _T), 0)
    oh_t = (krow == idxr[None, :]).astype(jnp.float32)
    cnt = jnp.dot(oh_t, jnp.ones((_T, 1), jnp.float32),
                  preferred_element_type=jnp.float32)

    y = jnp.dot(dec_ref[0], wpost_ref[...],
                preferred_element_type=jnp.float32)
    y_ref[0] = y + bpost_ref[...]

    s_mind = jnp.sum(r2) - 2.0 * jnp.sum(maxm)
    cm_ref[...] = (s_mind * (1.0 / (_T * _C))).reshape(1, 1, 1)
    p = cnt * (1.0 / _T)
    kl_ref[...] = jnp.sum(p * jnp.log(p * _K + 1e-10)).reshape(1, 1, 1)


def _sc_gather(cw_hbm, idx_hbm, out_hbm, idx_v, rows_v, sem):
    wid = lax.axis_index("s") * 2 + lax.axis_index("c")
    base = wid * _RW
    for c in range(_NCH):
        off = base + c * _CH
        pltpu.sync_copy(idx_hbm.at[pl.ds(off, _CH)], idx_v)
        pltpu.async_copy(cw_hbm.at[idx_v], rows_v, sem).wait()
        pltpu.sync_copy(rows_v, out_hbm.at[pl.ds(off, _CH)])


def _tc_add(y_ref, g_ref, out_ref):
    out_ref[0] = y_ref[0] + g_ref[0]


def kernel(enc, dec, W_pre, b_pre, W_post, b_post, codebook):
    bpre2 = b_pre.reshape(1, _C)
    bpost2 = b_post.reshape(1, _C)

    y3, idx3, cm3, kl3, cw = pl.pallas_call(
        _tc_a,
        grid=(_B,),
        in_specs=[
            pl.BlockSpec((1, _T, _C), lambda b: (b, 0, 0)),
            pl.BlockSpec((1, _T, _C), lambda b: (b, 0, 0)),
            pl.BlockSpec((_C, _C), lambda b: (0, 0)),
            pl.BlockSpec((1, _C), lambda b: (0, 0)),
            pl.BlockSpec((_C, _C), lambda b: (0, 0)),
            pl.BlockSpec((1, _C), lambda b: (0, 0)),
            pl.BlockSpec((_K, _C), lambda b: (0, 0)),
        ],
        out_specs=[
            pl.BlockSpec((1, _T, _C), lambda b: (b, 0, 0)),
            pl.BlockSpec((1, 1, _T), lambda b: (b, 0, 0)),
            pl.BlockSpec((1, 1, 1), lambda b: (b, 0, 0)),
            pl.BlockSpec((1, 1, 1), lambda b: (b, 0, 0)),
            pl.BlockSpec((_K, _C), lambda b: (0, 0)),
        ],
        out_shape=[
            jax.ShapeDtypeStruct((_B, _T, _C), jnp.float32),
            jax.ShapeDtypeStruct((_B, 1, _T), jnp.int32),
            jax.ShapeDtypeStruct((_B, 1, 1), jnp.float32),
            jax.ShapeDtypeStruct((_B, 1, 1), jnp.float32),
            jax.ShapeDtypeStruct((_K, _C), jnp.float32),
        ],
        scratch_shapes=[
            pltpu.VMEM((_K, _T), jnp.float32),
        ],
    )(enc, dec, W_pre, bpre2, W_post, bpost2, codebook)

    idx_flat = idx3.reshape(_B * _T)

    gather = functools.partial(
        pl.kernel,
        mesh=plsc.VectorSubcoreMesh(core_axis_name="c", subcore_axis_name="s"),
        out_type=jax.ShapeDtypeStruct((_B * _T, _C), jnp.float32),
        scratch_types=[
            pltpu.VMEM((_CH,), jnp.int32),
            pltpu.VMEM((_CH, _C), jnp.float32),
            pltpu.SemaphoreType.DMA,
        ],
    )(_sc_gather)
    gathered = gather(cw, idx_flat)

    out = pl.pallas_call(
        _tc_add,
        grid=(_B,),
        in_specs=[
            pl.BlockSpec((1, _T, _C), lambda b: (b, 0, 0)),
            pl.BlockSpec((1, _T, _C), lambda b: (b, 0, 0)),
        ],
        out_specs=pl.BlockSpec((1, _T, _C), lambda b: (b, 0, 0)),
        out_shape=jax.ShapeDtypeStruct((_B, _T, _C), jnp.float32),
    )(y3, gathered.reshape(_B, _T, _C))

    indices = idx3.reshape(_B, _T)
    cm = cm3.reshape(_B)
    kl = kl3.reshape(_B)
    return out, cm, cm, kl, indices


# speedup vs baseline: 1.9997x; 1.9997x over previous
import jax
import jax.numpy as jnp
from jax.experimental import pallas as pl
from jax.experimental.pallas import tpu as pltpu

_B, _T, _C, _K = 16, 2048, 256, 1024


def _fused_tc(enc_ref, dec_ref, wpre_ref, bpre_ref, wpost_ref, bpost_ref,
              cb_ref,
              out_ref, idx_ref, cm_ref, kl_ref,
              c2b_ref, cw_ref):
    b = pl.program_id(0)

    @pl.when(b == 0)
    def _init_consts():
        cb = cb_ref[...]
        c2col = jnp.sum(cb * cb, axis=1, keepdims=True)
        c2b_ref[...] = jnp.broadcast_to(c2col * 0.5, (_K, _T))
        cw_ref[...] = jnp.dot(cb, wpost_ref[...],
                              preferred_element_type=jnp.float32)

    x = enc_ref[0] - dec_ref[0]
    r = jnp.dot(x, wpre_ref[...],
                preferred_element_type=jnp.float32) + bpre_ref[...]
    dots_t = jax.lax.dot_general(cb_ref[...], r, (((1,), (1,)), ((), ())),
                                 preferred_element_type=jnp.float32)
    m_t = dots_t - c2b_ref[...]

    idxr = jnp.argmax(m_t, axis=0)
    maxm = jnp.max(m_t, axis=0, keepdims=True)
    rsq = r * r
    r2 = jnp.dot(rsq, jnp.ones((_C, 1), jnp.float32),
                 preferred_element_type=jnp.float32)
    idx_ref[0, 0, :] = idxr

    krow = jax.lax.broadcasted_iota(jnp.int32, (_K, _T), 0)
    oh_t = (krow == idxr[None, :]).astype(jnp.float32)
    cnt = jnp.dot(oh_t, jnp.ones((_T, 1), jnp.float32),
                  preferred_element_type=jnp.float32)

    quant = jax.lax.dot_general(oh_t, cw_ref[...], (((0,), (0,)), ((), ())),
                                preferred_element_type=jnp.float32)
    y = jnp.dot(dec_ref[0], wpost_ref[...],
                preferred_element_type=jnp.float32)
    out_ref[0] = y + quant + bpost_ref[...]

    s_mind = jnp.sum(r2) - 2.0 * jnp.sum(maxm)
    cm_ref[...] = (s_mind * (1.0 / (_T * _C))).reshape(1, 1, 1)
    p = cnt * (1.0 / _T)
    kl_ref[...] = jnp.sum(p * jnp.log(p * _K + 1e-10)).reshape(1, 1, 1)


def kernel(enc, dec, W_pre, b_pre, W_post, b_post, codebook):
    bpre2 = b_pre.reshape(1, _C)
    bpost2 = b_post.reshape(1, _C)

    out, idx3, cm3, kl3 = pl.pallas_call(
        _fused_tc,
        grid=(_B,),
        in_specs=[
            pl.BlockSpec((1, _T, _C), lambda b: (b, 0, 0)),
            pl.BlockSpec((1, _T, _C), lambda b: (b, 0, 0)),
            pl.BlockSpec((_C, _C), lambda b: (0, 0)),
            pl.BlockSpec((1, _C), lambda b: (0, 0)),
            pl.BlockSpec((_C, _C), lambda b: (0, 0)),
            pl.BlockSpec((1, _C), lambda b: (0, 0)),
            pl.BlockSpec((_K, _C), lambda b: (0, 0)),
        ],
        out_specs=[
            pl.BlockSpec((1, _T, _C), lambda b: (b, 0, 0)),
            pl.BlockSpec((1, 1, _T), lambda b: (b, 0, 0)),
            pl.BlockSpec((1, 1, 1), lambda b: (b, 0, 0)),
            pl.BlockSpec((1, 1, 1), lambda b: (b, 0, 0)),
        ],
        out_shape=[
            jax.ShapeDtypeStruct((_B, _T, _C), jnp.float32),
            jax.ShapeDtypeStruct((_B, 1, _T), jnp.int32),
            jax.ShapeDtypeStruct((_B, 1, 1), jnp.float32),
            jax.ShapeDtypeStruct((_B, 1, 1), jnp.float32),
        ],
        scratch_shapes=[
            pltpu.VMEM((_K, _T), jnp.float32),
            pltpu.VMEM((_K, _C), jnp.float32),
        ],
    )(enc, dec, W_pre, bpre2, W_post, bpost2, codebook)

    indices = idx3.reshape(_B, _T)
    cm = cm3.reshape(_B)
    kl = kl3.reshape(_B)
    return out, cm, cm, kl, indices
